# baseline (speedup 1.0000x reference)
import functools

import numpy as np
import jax
import jax.numpy as jnp
from jax import lax
from jax.experimental import pallas as pl
from jax.experimental.pallas import tpu as pltpu
from jax.experimental.pallas import tpu_sc as plsc

SEQ = 4096
DIM = 128
_LANES = 16
_NUM_CORES = 2
_NUM_SUBCORES = 16
_NW = _NUM_CORES * _NUM_SUBCORES
_B_PER_W = SEQ // _NW
_NCHUNK = 4
_CROWS = _B_PER_W // _NCHUNK


def _pe_table() -> np.ndarray:
    pos = np.arange(1, SEQ + 1, dtype=np.float64)[:, None]
    i = np.arange(1, DIM + 1, dtype=np.float64)[None, :]
    w_even = (1.0 / 10000.0) ** (i / DIM)
    w_odd = (1.0 / 10000.0) ** ((i - 1.0) / DIM)
    even = (np.arange(1, DIM + 1) % 2 == 0)[None, :]
    return np.where(even, np.sin(pos * w_even), np.cos(pos * w_odd)).astype(
        np.float32
    )


_PE_NP = _pe_table()

_mesh = plsc.VectorSubcoreMesh(core_axis_name="c", subcore_axis_name="s")


@functools.partial(
    pl.kernel,
    mesh=_mesh,
    out_type=jax.ShapeDtypeStruct((SEQ, DIM), jnp.float32),
    scratch_types=[
        pltpu.VMEM((_NCHUNK, _CROWS), jnp.int32),
        pltpu.VMEM((_B_PER_W, DIM), jnp.float32),
    ]
    + [pltpu.VMEM((_CROWS, DIM), jnp.float32) for _ in range(_NCHUNK)]
    + [pltpu.SemaphoreType.DMA for _ in range(_NCHUNK + 2)],
)
def _emb_pe_kernel(x_hbm, table_hbm, pe_hbm, out_hbm, idx_v, pe_v,
                   r0, r1, r2, r3, sg0, sg1, sg2, sg3, sem_p, sem_o):
    wid = lax.axis_index("s") * _NUM_CORES + lax.axis_index("c")
    base = wid * _B_PER_W
    rows = (r0, r1, r2, r3)
    sgs = (sg0, sg1, sg2, sg3)

    pltpu.sync_copy(x_hbm.at[wid], idx_v)
    pe_cp = pltpu.async_copy(pe_hbm.at[pl.ds(base, _B_PER_W)], pe_v, sem_p)
    gathers = [
        pltpu.async_copy(table_hbm.at[idx_v.at[k]], rows[k], sgs[k])
        for k in range(_NCHUNK)
    ]
    pe_cp.wait()

    outs = []
    for k in range(_NCHUNK):
        gathers[k].wait()
        rk = rows[k]

        def add_row(i, _, rk=rk, k=k):
            for j in range(DIM // _LANES):
                sl = pl.ds(j * _LANES, _LANES)
                plsc.addupdate(rk.at[i, sl], pe_v[k * _CROWS + i, sl])
            return ()

        lax.fori_loop(0, _CROWS, add_row, ())
        outs.append(
            pltpu.async_copy(
                rk, out_hbm.at[pl.ds(base + k * _CROWS, _CROWS)], sem_o
            )
        )
    for o in outs:
        o.wait()


def kernel(x, table):
    pe = jnp.asarray(_PE_NP)
    xw = x.astype(jnp.int32).reshape(_NW, _NCHUNK, _CROWS)
    return _emb_pe_kernel(xw, table, pe)
